# Initial kernel scaffold; baseline (speedup 1.0000x reference)
import jax
import jax.numpy as jnp
from jax.experimental import pallas as pl


def kernel(X, edges, E, W1, b1, W2, b2, W3, b3, W4, b4, Wp, bp, Wr1, br1, Wr2, br2, gate_scale):
    raise NotImplementedError("write your pallas kernel here")


# speedup vs baseline: 2.9038x; 2.9038x over previous
import functools

import jax
import jax.numpy as jnp
from jax import lax
from jax.experimental import pallas as pl
from jax.experimental.pallas import tpu as pltpu
from jax.experimental.pallas import tpu_sc as plsc

ND = 128
HID = 128
NB = 2000
EB = 2000
K = 128


def _stage1_body(x_ref, w1t_ref, b1_ref, w2t_ref, b2_ref, w3at_ref, w3bt_ref,
                 a_ref, b_ref, hxsum_ref):
    i = pl.program_id(0)
    x = x_ref[...]
    h = jnp.maximum(
        jnp.dot(x, w1t_ref[...], preferred_element_type=jnp.float32)
        + b1_ref[...], 0.0)
    hx = jnp.maximum(
        jnp.dot(h, w2t_ref[...], preferred_element_type=jnp.float32)
        + b2_ref[...], 0.0)
    a_ref[...] = jnp.dot(x, w3at_ref[...], preferred_element_type=jnp.float32)
    b_ref[...] = jnp.dot(x, w3bt_ref[...], preferred_element_type=jnp.float32)

    @pl.when(i == 0)
    def _():
        hxsum_ref[...] = jnp.zeros_like(hxsum_ref)

    hxsum_ref[...] += jnp.sum(hx, axis=0, keepdims=True)


def _stage1(X, W1t, b1, W2t, b2, W3at, W3bt):
    n = X.shape[0]
    grid = n // NB
    full = lambda i: (0, 0)
    return pl.pallas_call(
        _stage1_body,
        grid=(grid,),
        in_specs=[
            pl.BlockSpec((NB, ND), lambda i: (i, 0)),
            pl.BlockSpec((ND, HID), full),
            pl.BlockSpec((1, HID), full),
            pl.BlockSpec((HID, HID), full),
            pl.BlockSpec((1, HID), full),
            pl.BlockSpec((ND, HID), full),
            pl.BlockSpec((ND, HID), full),
        ],
        out_specs=[
            pl.BlockSpec((NB, HID), lambda i: (i, 0)),
            pl.BlockSpec((NB, HID), lambda i: (i, 0)),
            pl.BlockSpec((1, HID), full),
        ],
        out_shape=[
            jax.ShapeDtypeStruct((n, HID), jnp.float32),
            jax.ShapeDtypeStruct((n, HID), jnp.float32),
            jax.ShapeDtypeStruct((1, HID), jnp.float32),
        ],
    )(X, W1t, b1, W2t, b2, W3at, W3bt)


def _stage2(A, B, srcp, dstp):
    info = plsc.get_sparse_core_info()
    nc, ns = info.num_cores, info.num_subcores
    nw = nc * ns
    nep = srcp.shape[0]
    epw = nep // nw
    nch = epw // K

    mesh = plsc.VectorSubcoreMesh(core_axis_name="c", subcore_axis_name="s")

    @functools.partial(
        pl.kernel,
        mesh=mesh,
        out_type=(jax.ShapeDtypeStruct((nep, HID), jnp.float32),
                  jax.ShapeDtypeStruct((nep, HID), jnp.float32)),
        scratch_types=[
            pltpu.VMEM((K,), jnp.int32),
            pltpu.VMEM((K,), jnp.int32),
            pltpu.VMEM((K, HID), jnp.float32),
            pltpu.VMEM((K, HID), jnp.float32),
            pltpu.SemaphoreType.DMA,
            pltpu.SemaphoreType.DMA,
        ],
    )
    def gather_kernel(a_hbm, b_hbm, src_hbm, dst_hbm, ga_hbm, gb_hbm,
                      si_v, di_v, ra_v, rb_v, sema, semb):
        wid = lax.axis_index("s") * nc + lax.axis_index("c")
        base = wid * epw

        def body(j, carry):
            off = base + j * K
            pltpu.sync_copy(src_hbm.at[pl.ds(off, K)], si_v)
            pltpu.sync_copy(dst_hbm.at[pl.ds(off, K)], di_v)
            ca = pltpu.async_copy(a_hbm.at[si_v], ra_v, sema)
            cb = pltpu.async_copy(b_hbm.at[di_v], rb_v, semb)
            ca.wait()
            cb.wait()
            pltpu.sync_copy(ra_v, ga_hbm.at[pl.ds(off, K)])
            pltpu.sync_copy(rb_v, gb_hbm.at[pl.ds(off, K)])
            return carry

        lax.fori_loop(0, nch, body, 0)

    return gather_kernel(A, B, srcp, dstp)


def _stage3_body(ga_ref, gb_ref, e_ref, w3ct_ref, b3_ref, w4t_ref, b4_ref,
                 wpt_ref, bp_ref, gs_ref, msum_ref, ctx_ref):
    i = pl.program_id(0)
    e = e_ref[...]
    gate = jnp.clip(1.0 + gs_ref[0, 0] * e[:, 2:3], 0.0, 3.0)
    h1 = jnp.maximum(
        ga_ref[...] + gb_ref[...]
        + jnp.dot(e, w3ct_ref[...], preferred_element_type=jnp.float32)
        + b3_ref[...], 0.0)
    m = jnp.maximum(
        jnp.dot(h1, w4t_ref[...], preferred_element_type=jnp.float32)
        + b4_ref[...], 0.0) * gate
    p = (jnp.dot(e, wpt_ref[...], preferred_element_type=jnp.float32)
         + bp_ref[...]) * gate

    @pl.when(i == 0)
    def _():
        msum_ref[...] = jnp.zeros_like(msum_ref)
        ctx_ref[...] = jnp.zeros_like(ctx_ref)

    msum_ref[...] += jnp.sum(m, axis=0, keepdims=True)
    ctx_ref[...] += jnp.sum(p, axis=0, keepdims=True)


def _stage3(GA, GB, E, W3ct, b3, W4t, b4, Wpt, bp, gs):
    ne = E.shape[0]
    grid = ne // EB
    full = lambda i: (0, 0)
    return pl.pallas_call(
        _stage3_body,
        grid=(grid,),
        in_specs=[
            pl.BlockSpec((EB, HID), lambda i: (i, 0)),
            pl.BlockSpec((EB, HID), lambda i: (i, 0)),
            pl.BlockSpec((EB, 4), lambda i: (i, 0)),
            pl.BlockSpec((4, HID), full),
            pl.BlockSpec((1, HID), full),
            pl.BlockSpec((HID, HID), full),
            pl.BlockSpec((1, HID), full),
            pl.BlockSpec((4, HID), full),
            pl.BlockSpec((1, HID), full),
            pl.BlockSpec((1, 1), full),
        ],
        out_specs=[
            pl.BlockSpec((1, HID), full),
            pl.BlockSpec((1, HID), full),
        ],
        out_shape=[
            jax.ShapeDtypeStruct((1, HID), jnp.float32),
            jax.ShapeDtypeStruct((1, HID), jnp.float32),
        ],
    )(GA, GB, E, W3ct, b3, W4t, b4, Wpt, bp, gs)


def _stage4_body(hxsum_ref, msum_ref, ctxsum_ref, wr1at_ref, wr1bt_ref,
                 br1_ref, wr2t_ref, br2_ref, out_ref, *, inv_n, inv_ne):
    hmean = (hxsum_ref[...] + 2.0 * msum_ref[...]) * inv_n
    ctx = ctxsum_ref[...] * inv_ne
    h = jnp.maximum(
        jnp.dot(hmean, wr1at_ref[...], preferred_element_type=jnp.float32)
        + jnp.dot(ctx, wr1bt_ref[...], preferred_element_type=jnp.float32)
        + br1_ref[...], 0.0)
    out_ref[...] = (jnp.dot(h, wr2t_ref[...],
                            preferred_element_type=jnp.float32) + br2_ref[...])


def _stage4(hxsum, msum, ctxsum, Wr1at, Wr1bt, br1, Wr2t, br2, n, ne):
    body = functools.partial(_stage4_body, inv_n=1.0 / n,
                             inv_ne=1.0 / (ne + 1e-06))
    return pl.pallas_call(
        body,
        out_shape=jax.ShapeDtypeStruct((1, 2), jnp.float32),
    )(hxsum, msum, ctxsum, Wr1at, Wr1bt, br1, Wr2t, br2)


def kernel(X, edges, E, W1, b1, W2, b2, W3, b3, W4, b4, Wp, bp,
           Wr1, br1, Wr2, br2, gate_scale):
    n = X.shape[0]
    ne = edges.shape[0]

    W1t = W1.T
    W2t = W2.T
    W3at = W3[:, :ND].T
    W3bt = W3[:, ND:2 * ND].T
    W3ct = W3[:, 2 * ND:].T
    W4t = W4.T
    Wpt = Wp.T
    Wr1at = Wr1[:, :HID].T
    Wr1bt = Wr1[:, HID:].T
    Wr2t = Wr2.T
    b1r = b1[None, :]
    b2r = b2[None, :]
    b3r = b3[None, :]
    b4r = b4[None, :]
    bpr = bp[None, :]
    br1r = br1[None, :]
    br2r = br2[None, :]
    gs = jnp.reshape(gate_scale.astype(jnp.float32), (1, 1))

    unit = 32 * K
    nep = ((ne + unit - 1) // unit) * unit
    src = edges[:, 0]
    dst = edges[:, 1]
    pad = jnp.zeros((nep - ne,), jnp.int32)
    srcp = jnp.concatenate([src, pad])
    dstp = jnp.concatenate([dst, pad])

    A, B, hxsum = _stage1(X, W1t, b1r, W2t, b2r, W3at, W3bt)
    GA, GB = _stage2(A, B, srcp, dstp)
    msum, ctxsum = _stage3(GA, GB, E, W3ct, b3r, W4t, b4r, Wpt, bpr, gs)
    return _stage4(hxsum, msum, ctxsum, Wr1at, Wr1bt, br1r, Wr2t, br2r, n, ne)
